# baseline (speedup 1.0000x reference)
import jax
import jax.numpy as jnp
import numpy as np
from jax import lax
from jax.experimental import pallas as pl
from jax.experimental.pallas import tpu as pltpu
from jax.experimental.pallas import tpu_sc as plsc

N_POINTS = 262144
N_LEVELS = 10
F_PER_LEVEL = 8
ENC_DIM = N_LEVELS * F_PER_LEVEL
BASE_RES = 16
MAX_RES = 1024
LOG2_HASH = 19
HASHMAP_SIZE = 2 ** LOG2_HASH
PER_LEVEL_SCALE = float(np.exp((np.log(MAX_RES) - np.log(BASE_RES)) / (N_LEVELS - 1)))
PRIME_I32 = np.int32(np.uint32(2654435761).view(np.int32))
HASH_MASK = np.int32(HASHMAP_SIZE - 1)

LEVEL_META = []
for _l in range(N_LEVELS):
    _scale = BASE_RES * (PER_LEVEL_SCALE ** _l) - 1.0
    _res = int(np.ceil(_scale)) + 1
    LEVEL_META.append((_scale, _res, (_res * _res) > HASHMAP_SIZE, _l * HASHMAP_SIZE))

NUM_CORES = 2
NUM_SUBCORES = 16
NW = NUM_CORES * NUM_SUBCORES
PTS_PER_W = N_POINTS // NW
CHUNK = 512
N_CHUNKS = PTS_PER_W // CHUNK
NGRP = CHUNK // 128


def _sc_encode_body(coords_hbm, tab_hbm, out_hbm,
                    cb, idx, fx, fy, rb, encb, semg):
    c = lax.axis_index("c")
    s = lax.axis_index("s")
    wid = s * NUM_CORES + c
    base0 = wid * PTS_PER_W

    lanes = lax.iota(jnp.int32, 16)
    half = lanes >> 3
    col8 = lanes & 7
    zero16 = lanes * 0
    one16 = zero16 + 1

    def compute_indices(l, p):
        scale, res, hashed, off = LEVEL_META[l]

        @pl.loop(0, CHUNK // 16)
        def vloop(v):
            sl = pl.ds(v * 16, 16)
            rowv = lax.iota(jnp.int32, 16) + v * 16
            x = plsc.load_gather(cb, [rowv, zero16])
            y = plsc.load_gather(cb, [rowv, one16])
            px = x * scale + 0.5
            py = y * scale + 0.5
            gx = px.astype(jnp.int32)
            gy = py.astype(jnp.int32)
            fx[p][sl] = px - gx.astype(jnp.float32)
            fy[p][sl] = py - gy.astype(jnp.float32)
            g = v // 8
            sg = pl.ds((v % 8) * 16, 16)
            if hashed:
                t0 = gy * PRIME_I32
                t1 = (gy + 1) * PRIME_I32
                idx[p][0][g, sg] = ((gx ^ t0) & HASH_MASK) + off
                idx[p][1][g, sg] = (((gx + 1) ^ t0) & HASH_MASK) + off
                idx[p][2][g, sg] = ((gx ^ t1) & HASH_MASK) + off
                idx[p][3][g, sg] = (((gx + 1) ^ t1) & HASH_MASK) + off
            else:
                v00 = gx + gy * res + off
                idx[p][0][g, sg] = v00
                idx[p][1][g, sg] = v00 + 1
                idx[p][2][g, sg] = v00 + res
                idx[p][3][g, sg] = v00 + (res + 1)

    def fire_gathers(p):
        descs = []
        for corner in range(4):
            for g in range(NGRP):
                descs.append(pltpu.async_copy(
                    tab_hbm.at[idx[p][corner].at[g]],
                    rb[p][corner].at[pl.ds(g * 128, 128), :],
                    semg[p]))
        return descs

    def combine(l, p):
        col_l = col8 + 8 * l

        @pl.loop(0, CHUNK // 2)
        def ploop(j):
            rowv = half + 2 * j
            fxv = plsc.load_gather(fx[p], [rowv])
            fyv = plsc.load_gather(fy[p], [rowv])
            a00 = plsc.load_gather(rb[p][0], [rowv, col8])
            a10 = plsc.load_gather(rb[p][1], [rowv, col8])
            a01 = plsc.load_gather(rb[p][2], [rowv, col8])
            a11 = plsc.load_gather(rb[p][3], [rowv, col8])
            ax0 = a00 + fxv * (a10 - a00)
            ax1 = a01 + fxv * (a11 - a01)
            plsc.store_scatter(encb, [rowv, col_l], ax0 + fyv * (ax1 - ax0))

    @pl.loop(0, N_CHUNKS)
    def chunk_loop(ch):
        base = base0 + ch * CHUNK
        pltpu.sync_copy(coords_hbm.at[pl.ds(base, CHUNK), :], cb)

        compute_indices(0, 0)
        descs = fire_gathers(0)
        for l in range(N_LEVELS):
            p = l % 2
            if l < N_LEVELS - 1:
                compute_indices(l + 1, 1 - p)
                next_descs = fire_gathers(1 - p)
            else:
                next_descs = []
            for d in descs:
                d.wait()
            combine(l, p)
            descs = next_descs

        pltpu.sync_copy(encb, out_hbm.at[pl.ds(base, CHUNK), :])


@jax.jit
def _sc_encode(coords, tab_flat):
    mesh = plsc.VectorSubcoreMesh(core_axis_name="c", subcore_axis_name="s")
    f = pl.kernel(
        _sc_encode_body,
        out_type=jax.ShapeDtypeStruct((N_POINTS, ENC_DIM), jnp.float32),
        mesh=mesh,
        compiler_params=pltpu.CompilerParams(
            needs_layout_passes=False, use_tc_tiling_on_sc=False),
        scratch_types=[
            pltpu.VMEM((CHUNK, 2), jnp.float32),
            [[pltpu.VMEM((NGRP, 128), jnp.int32) for _ in range(4)]
             for _ in range(2)],
            [pltpu.VMEM((CHUNK,), jnp.float32) for _ in range(2)],
            [pltpu.VMEM((CHUNK,), jnp.float32) for _ in range(2)],
            [[pltpu.VMEM((CHUNK, F_PER_LEVEL), jnp.float32) for _ in range(4)]
             for _ in range(2)],
            pltpu.VMEM((CHUNK, ENC_DIM), jnp.float32),
            [pltpu.SemaphoreType.DMA for _ in range(2)],
        ],
    )
    return f(coords, tab_flat)


BN = 4096


def _mlp_body(enc_ref, w1_ref, b1_ref, w2_ref, b2_ref, out_ref):
    h = jnp.dot(enc_ref[...], w1_ref[...], preferred_element_type=jnp.float32)
    h = jnp.maximum(h + b1_ref[...], 0.0)
    out_ref[...] = jnp.dot(h, w2_ref[...],
                           preferred_element_type=jnp.float32) + b2_ref[...]


@jax.jit
def _tc_mlp(enc, w1, b1r, w2, b2r):
    return pl.pallas_call(
        _mlp_body,
        grid=(N_POINTS // BN,),
        in_specs=[
            pl.BlockSpec((BN, ENC_DIM), lambda i: (i, 0)),
            pl.BlockSpec((ENC_DIM, 64), lambda i: (0, 0)),
            pl.BlockSpec((1, 64), lambda i: (0, 0)),
            pl.BlockSpec((64, 128), lambda i: (0, 0)),
            pl.BlockSpec((1, 128), lambda i: (0, 0)),
        ],
        out_specs=pl.BlockSpec((BN, 128), lambda i: (i, 0)),
        out_shape=jax.ShapeDtypeStruct((N_POINTS, 128), jnp.float32),
    )(enc, w1, b1r, w2, b2r)


def kernel(coords, table, W1, b1, W2, b2):
    tab_flat = table.reshape(N_LEVELS * HASHMAP_SIZE, F_PER_LEVEL)
    enc = _sc_encode(coords, tab_flat)
    return jnp.pad(enc, ((0, 0), (0, 48)))
    return _tc_mlp(enc, W1, b1.reshape(1, 64), W2, b2.reshape(1, 128))


# speedup vs baseline: 1.0572x; 1.0572x over previous
import jax
import jax.numpy as jnp
import numpy as np
from jax import lax
from jax.experimental import pallas as pl
from jax.experimental.pallas import tpu as pltpu
from jax.experimental.pallas import tpu_sc as plsc

N_POINTS = 262144
N_LEVELS = 10
F_PER_LEVEL = 8
ENC_DIM = N_LEVELS * F_PER_LEVEL
BASE_RES = 16
MAX_RES = 1024
LOG2_HASH = 19
HASHMAP_SIZE = 2 ** LOG2_HASH
PER_LEVEL_SCALE = float(np.exp((np.log(MAX_RES) - np.log(BASE_RES)) / (N_LEVELS - 1)))
PRIME_I32 = np.int32(np.uint32(2654435761).view(np.int32))
HASH_MASK = np.int32(HASHMAP_SIZE - 1)

LEVEL_META = []
for _l in range(N_LEVELS):
    _scale = BASE_RES * (PER_LEVEL_SCALE ** _l) - 1.0
    _res = int(np.ceil(_scale)) + 1
    LEVEL_META.append((_scale, _res, (_res * _res) > HASHMAP_SIZE, _l * HASHMAP_SIZE))

NUM_CORES = 2
NUM_SUBCORES = 16
NW = NUM_CORES * NUM_SUBCORES
PTS_PER_W = N_POINTS // NW
CHUNK = 512
N_CHUNKS = PTS_PER_W // CHUNK
NGRP = CHUNK // 128


def _sc_encode_body(coords_hbm, tab_hbm, out_hbm,
                    cb, idx, fx, fy, rb, encb, semg):
    c = lax.axis_index("c")
    s = lax.axis_index("s")
    wid = s * NUM_CORES + c
    base0 = wid * PTS_PER_W

    lanes = lax.iota(jnp.int32, 16)
    half = lanes >> 3
    col8 = lanes & 7
    zero16 = lanes * 0
    one16 = zero16 + 1

    @pl.loop(0, CHUNK)
    def zero_pad_loop(r):
        z = jnp.zeros((16,), jnp.float32)
        encb[r, pl.ds(ENC_DIM, 16)] = z
        encb[r, pl.ds(ENC_DIM + 16, 16)] = z
        encb[r, pl.ds(ENC_DIM + 32, 16)] = z

    def compute_indices(l, p):
        scale, res, hashed, off = LEVEL_META[l]

        @pl.loop(0, CHUNK // 16)
        def vloop(v):
            sl = pl.ds(v * 16, 16)
            rowv = lax.iota(jnp.int32, 16) + v * 16
            x = plsc.load_gather(cb, [rowv, zero16])
            y = plsc.load_gather(cb, [rowv, one16])
            px = x * scale + 0.5
            py = y * scale + 0.5
            gx = px.astype(jnp.int32)
            gy = py.astype(jnp.int32)
            fx[p][sl] = px - gx.astype(jnp.float32)
            fy[p][sl] = py - gy.astype(jnp.float32)
            g = v // 8
            sg = pl.ds((v % 8) * 16, 16)
            if hashed:
                t0 = gy * PRIME_I32
                t1 = (gy + 1) * PRIME_I32
                idx[p][0][g, sg] = (gx ^ t0) & HASH_MASK
                idx[p][1][g, sg] = ((gx + 1) ^ t0) & HASH_MASK
                idx[p][2][g, sg] = (gx ^ t1) & HASH_MASK
                idx[p][3][g, sg] = ((gx + 1) ^ t1) & HASH_MASK
            else:
                v00 = gx + gy * res
                idx[p][0][g, sg] = v00
                idx[p][1][g, sg] = v00 + 1
                idx[p][2][g, sg] = v00 + res
                idx[p][3][g, sg] = v00 + (res + 1)

    def fire_gathers(l, p):
        descs = []
        for corner in range(4):
            for g in range(NGRP):
                descs.append(pltpu.async_copy(
                    tab_hbm.at[l].at[idx[p][corner].at[g]],
                    rb[p][corner].at[pl.ds(g * 128, 128), :],
                    semg[p]))
        return descs

    def combine(l, p):
        col_l = col8 + 8 * l

        @pl.loop(0, CHUNK // 2)
        def ploop(j):
            rowv = half + 2 * j
            fxv = plsc.load_gather(fx[p], [rowv])
            fyv = plsc.load_gather(fy[p], [rowv])
            a00 = plsc.load_gather(rb[p][0], [rowv, col8])
            a10 = plsc.load_gather(rb[p][1], [rowv, col8])
            a01 = plsc.load_gather(rb[p][2], [rowv, col8])
            a11 = plsc.load_gather(rb[p][3], [rowv, col8])
            ax0 = a00 + fxv * (a10 - a00)
            ax1 = a01 + fxv * (a11 - a01)
            plsc.store_scatter(encb, [rowv, col_l], ax0 + fyv * (ax1 - ax0))

    @pl.loop(0, N_CHUNKS)
    def chunk_loop(ch):
        base = base0 + ch * CHUNK
        pltpu.sync_copy(coords_hbm.at[pl.ds(base, CHUNK), :], cb)

        compute_indices(0, 0)
        descs = fire_gathers(0, 0)
        for l in range(N_LEVELS):
            p = l % 2
            if l < N_LEVELS - 1:
                compute_indices(l + 1, 1 - p)
                next_descs = fire_gathers(l + 1, 1 - p)
            else:
                next_descs = []
            for d in descs:
                d.wait()
            combine(l, p)
            descs = next_descs

        pltpu.sync_copy(encb, out_hbm.at[pl.ds(base, CHUNK), :])


@jax.jit
def _sc_encode(coords, table):
    mesh = plsc.VectorSubcoreMesh(core_axis_name="c", subcore_axis_name="s")
    f = pl.kernel(
        _sc_encode_body,
        out_type=jax.ShapeDtypeStruct((N_POINTS, 128), jnp.float32),
        mesh=mesh,
        compiler_params=pltpu.CompilerParams(
            needs_layout_passes=False, use_tc_tiling_on_sc=False),
        scratch_types=[
            pltpu.VMEM((CHUNK, 2), jnp.float32),
            [[pltpu.VMEM((NGRP, 128), jnp.int32) for _ in range(4)]
             for _ in range(2)],
            [pltpu.VMEM((CHUNK,), jnp.float32) for _ in range(2)],
            [pltpu.VMEM((CHUNK,), jnp.float32) for _ in range(2)],
            [[pltpu.VMEM((CHUNK, F_PER_LEVEL), jnp.float32) for _ in range(4)]
             for _ in range(2)],
            pltpu.VMEM((CHUNK, 128), jnp.float32),
            [pltpu.SemaphoreType.DMA for _ in range(2)],
        ],
    )
    return f(coords, table)


BN = 4096


def _mlp_body(enc_ref, w1_ref, b1_ref, w2_ref, b2_ref, out_ref):
    h = jnp.dot(enc_ref[...], w1_ref[...], preferred_element_type=jnp.float32)
    h = jnp.maximum(h + b1_ref[...], 0.0)
    out_ref[...] = jnp.dot(h, w2_ref[...],
                           preferred_element_type=jnp.float32) + b2_ref[...]


@jax.jit
def _tc_mlp(enc, w1, b1r, w2, b2r):
    return pl.pallas_call(
        _mlp_body,
        grid=(N_POINTS // BN,),
        in_specs=[
            pl.BlockSpec((BN, 128), lambda i: (i, 0)),
            pl.BlockSpec((128, 64), lambda i: (0, 0)),
            pl.BlockSpec((1, 64), lambda i: (0, 0)),
            pl.BlockSpec((64, 128), lambda i: (0, 0)),
            pl.BlockSpec((1, 128), lambda i: (0, 0)),
        ],
        out_specs=pl.BlockSpec((BN, 128), lambda i: (i, 0)),
        out_shape=jax.ShapeDtypeStruct((N_POINTS, 128), jnp.float32),
    )(enc, w1, b1r, w2, b2r)


def kernel(coords, table, W1, b1, W2, b2):
    enc = _sc_encode(coords, table)
    w1p = jnp.pad(W1, ((0, 128 - ENC_DIM), (0, 0)))
    return _tc_mlp(enc, w1p, b1.reshape(1, 64), W2, b2.reshape(1, 128))


# speedup vs baseline: 1.8361x; 1.7368x over previous
import jax
import jax.numpy as jnp
import numpy as np
from jax import lax
from jax.experimental import pallas as pl
from jax.experimental.pallas import tpu as pltpu
from jax.experimental.pallas import tpu_sc as plsc

N_POINTS = 262144
N_LEVELS = 10
F_PER_LEVEL = 8
ENC_DIM = N_LEVELS * F_PER_LEVEL
BASE_RES = 16
MAX_RES = 1024
LOG2_HASH = 19
HASHMAP_SIZE = 2 ** LOG2_HASH
PER_LEVEL_SCALE = float(np.exp((np.log(MAX_RES) - np.log(BASE_RES)) / (N_LEVELS - 1)))
PRIME_I32 = np.int32(np.uint32(2654435761).view(np.int32))
HASH_MASK = np.int32(HASHMAP_SIZE - 1)

LEVEL_META = []
TAB_SIZES = []
_off = 0
for _l in range(N_LEVELS):
    _scale = BASE_RES * (PER_LEVEL_SCALE ** _l) - 1.0
    _res = int(np.ceil(_scale)) + 1
    _hashed = (_res * _res) > HASHMAP_SIZE
    _n = HASHMAP_SIZE if _hashed else min(_res * _res + _res + 1, HASHMAP_SIZE)
    LEVEL_META.append((_scale, _res, _hashed, _off))
    TAB_SIZES.append(_n)
    _off += _n
TAB_ROWS = _off

NUM_CORES = 2
NUM_SUBCORES = 16
NW = NUM_CORES * NUM_SUBCORES
PTS_PER_W = N_POINTS // NW
CHUNK = 512
N_CHUNKS = PTS_PER_W // CHUNK
NGRP = CHUNK // 128


def _sc_encode_body(coords_hbm, tab_hbm, out_hbm,
                    cb, idx, fx, fy, rb, encb, semg):
    c = lax.axis_index("c")
    s = lax.axis_index("s")
    wid = s * NUM_CORES + c
    base0 = wid * PTS_PER_W

    lanes = lax.iota(jnp.int32, 16)
    half = lanes >> 3
    col8 = lanes & 7
    zero16 = lanes * 0
    one16 = zero16 + 1

    @pl.loop(0, CHUNK)
    def zero_pad_loop(r):
        z = jnp.zeros((16,), jnp.float32)
        encb[r, pl.ds(ENC_DIM, 16)] = z
        encb[r, pl.ds(ENC_DIM + 16, 16)] = z
        encb[r, pl.ds(ENC_DIM + 32, 16)] = z

    def compute_indices(l, p):
        scale, res, hashed, off = LEVEL_META[l]

        @pl.loop(0, CHUNK // 16)
        def vloop(v):
            sl = pl.ds(v * 16, 16)
            rowv = lax.iota(jnp.int32, 16) + v * 16
            x = plsc.load_gather(cb, [rowv, zero16])
            y = plsc.load_gather(cb, [rowv, one16])
            px = x * scale + 0.5
            py = y * scale + 0.5
            gx = px.astype(jnp.int32)
            gy = py.astype(jnp.int32)
            fx[p][sl] = px - gx.astype(jnp.float32)
            fy[p][sl] = py - gy.astype(jnp.float32)
            g = v // 8
            sg = pl.ds((v % 8) * 16, 16)
            if hashed:
                t0 = gy * PRIME_I32
                t1 = (gy + 1) * PRIME_I32
                idx[p][0][g, sg] = ((gx ^ t0) & HASH_MASK) + off
                idx[p][1][g, sg] = (((gx + 1) ^ t0) & HASH_MASK) + off
                idx[p][2][g, sg] = ((gx ^ t1) & HASH_MASK) + off
                idx[p][3][g, sg] = (((gx + 1) ^ t1) & HASH_MASK) + off
            else:
                v00 = gx + gy * res + off
                idx[p][0][g, sg] = v00
                idx[p][1][g, sg] = v00 + 1
                idx[p][2][g, sg] = v00 + res
                idx[p][3][g, sg] = v00 + (res + 1)

    def fire_gathers(l, p):
        descs = []
        for corner in range(4):
            for g in range(NGRP):
                descs.append(pltpu.async_copy(
                    tab_hbm.at[idx[p][corner].at[g]],
                    rb[p][corner].at[pl.ds(g * 128, 128), :],
                    semg[p]))
        return descs

    def combine(l, p):
        col_l = col8 + 8 * l

        @pl.loop(0, CHUNK // 2)
        def ploop(j):
            rowv = half + 2 * j
            fxv = plsc.load_gather(fx[p], [rowv])
            fyv = plsc.load_gather(fy[p], [rowv])
            a00 = plsc.load_gather(rb[p][0], [rowv, col8])
            a10 = plsc.load_gather(rb[p][1], [rowv, col8])
            a01 = plsc.load_gather(rb[p][2], [rowv, col8])
            a11 = plsc.load_gather(rb[p][3], [rowv, col8])
            ax0 = a00 + fxv * (a10 - a00)
            ax1 = a01 + fxv * (a11 - a01)
            plsc.store_scatter(encb, [rowv, col_l], ax0 + fyv * (ax1 - ax0))

    @pl.loop(0, N_CHUNKS)
    def chunk_loop(ch):
        base = base0 + ch * CHUNK
        pltpu.sync_copy(coords_hbm.at[pl.ds(base, CHUNK), :], cb)

        compute_indices(0, 0)
        descs = fire_gathers(0, 0)
        for l in range(N_LEVELS):
            p = l % 2
            if l < N_LEVELS - 1:
                compute_indices(l + 1, 1 - p)
                next_descs = fire_gathers(l + 1, 1 - p)
            else:
                next_descs = []
            for d in descs:
                d.wait()
            combine(l, p)
            descs = next_descs

        pltpu.sync_copy(encb, out_hbm.at[pl.ds(base, CHUNK), :])


@jax.jit
def _sc_encode(coords, table):
    mesh = plsc.VectorSubcoreMesh(core_axis_name="c", subcore_axis_name="s")
    f = pl.kernel(
        _sc_encode_body,
        out_type=jax.ShapeDtypeStruct((N_POINTS, 128), jnp.float32),
        mesh=mesh,
        compiler_params=pltpu.CompilerParams(
            needs_layout_passes=False, use_tc_tiling_on_sc=False),
        scratch_types=[
            pltpu.VMEM((CHUNK, 2), jnp.float32),
            [[pltpu.VMEM((NGRP, 128), jnp.int32) for _ in range(4)]
             for _ in range(2)],
            [pltpu.VMEM((CHUNK,), jnp.float32) for _ in range(2)],
            [pltpu.VMEM((CHUNK,), jnp.float32) for _ in range(2)],
            [[pltpu.VMEM((CHUNK, F_PER_LEVEL), jnp.float32) for _ in range(4)]
             for _ in range(2)],
            pltpu.VMEM((CHUNK, 128), jnp.float32),
            [pltpu.SemaphoreType.DMA for _ in range(2)],
        ],
    )
    return f(coords, table)


BN = 4096


def _mlp_body(enc_ref, w1_ref, b1_ref, w2_ref, b2_ref, out_ref):
    h = jnp.dot(enc_ref[...], w1_ref[...], preferred_element_type=jnp.float32)
    h = jnp.maximum(h + b1_ref[...], 0.0)
    out_ref[...] = jnp.dot(h, w2_ref[...],
                           preferred_element_type=jnp.float32) + b2_ref[...]


@jax.jit
def _tc_mlp(enc, w1, b1r, w2, b2r):
    return pl.pallas_call(
        _mlp_body,
        grid=(N_POINTS // BN,),
        in_specs=[
            pl.BlockSpec((BN, 128), lambda i: (i, 0)),
            pl.BlockSpec((128, 64), lambda i: (0, 0)),
            pl.BlockSpec((1, 64), lambda i: (0, 0)),
            pl.BlockSpec((64, 128), lambda i: (0, 0)),
            pl.BlockSpec((1, 128), lambda i: (0, 0)),
        ],
        out_specs=pl.BlockSpec((BN, 128), lambda i: (i, 0)),
        out_shape=jax.ShapeDtypeStruct((N_POINTS, 128), jnp.float32),
    )(enc, w1, b1r, w2, b2r)


def kernel(coords, table, W1, b1, W2, b2):
    tab_used = jnp.concatenate(
        [lax.slice_in_dim(table, l, l + 1, axis=0).reshape(HASHMAP_SIZE,
                                                           F_PER_LEVEL)[:n]
         for l, n in enumerate(TAB_SIZES)], axis=0)
    enc = _sc_encode(coords, tab_used)
    w1p = jnp.pad(W1, ((0, 128 - ENC_DIM), (0, 0)))
    return _tc_mlp(enc, w1p, b1.reshape(1, 64), W2, b2.reshape(1, 128))
